# baseline (speedup 1.0000x reference)
import functools

import jax
import jax.numpy as jnp
from jax import lax
from jax.experimental import pallas as pl
from jax.experimental.pallas import tpu as pltpu
from jax.experimental.pallas import tpu_sc as plsc

D = 128
R, S = 4096, 50
NC, NS = 2, 16
NW = NC * NS
R_PER_W = R // NW
NBUF = 3
LA = 2
NSLOT = 2


def _emb_body(xt_hbm, table_hbm, out_hbm, idx_v, b0, b1, b2, shr,
              isem, g0, g1, g2, s0, s1, d0, d1):
    sid = lax.axis_index("s")
    wid = sid * NC + lax.axis_index("c")
    r0 = wid * R_PER_W
    pltpu.async_copy(
        xt_hbm.at[pl.ds(wid * S * R_PER_W, S * R_PER_W)], idx_v, isem).wait()

    bufs = (b0, b1, b2)
    gsem = (g0, g1, g2)
    ssem = (s0, s1)
    dsem = (d0, d1)

    def gather(s):
        return pltpu.async_copy(
            table_hbm.at[idx_v.at[pl.ds(s * R_PER_W, R_PER_W)]],
            bufs[s % NBUF], gsem[s % NBUF])

    gathers = [None] * S
    dmas = [None] * S
    for s in range(LA):
        gathers[s] = gather(s)
    for s in range(S):
        gathers[s].wait()
        j = s + LA
        if j < S:
            gathers[j] = gather(j)
        slot = s % NSLOT
        if s - NSLOT >= 0:
            dmas[s - NSLOT].wait()
        pltpu.async_copy(bufs[s % NBUF], shr.at[slot, sid], ssem[slot]).wait()
        dmas[s] = pltpu.async_copy(
            shr.at[slot, sid], out_hbm.at[s, pl.ds(r0, R_PER_W)], dsem[slot])
    for s in range(S - NSLOT, S):
        dmas[s].wait()


_emb = functools.partial(
    pl.kernel,
    out_type=jax.ShapeDtypeStruct((S, R, D), jnp.float32),
    mesh=plsc.VectorSubcoreMesh(core_axis_name="c", subcore_axis_name="s"),
    scratch_types=[
        pltpu.VMEM((S * R_PER_W,), jnp.int32),
        pltpu.VMEM((R_PER_W, D), jnp.float32),
        pltpu.VMEM((R_PER_W, D), jnp.float32),
        pltpu.VMEM((R_PER_W, D), jnp.float32),
        pltpu.VMEM_SHARED((NSLOT, NS, R_PER_W, D), jnp.float32),
        pltpu.SemaphoreType.DMA,
        pltpu.SemaphoreType.DMA,
        pltpu.SemaphoreType.DMA,
        pltpu.SemaphoreType.DMA,
        pltpu.SemaphoreType.DMA,
        pltpu.SemaphoreType.DMA,
        pltpu.SemaphoreType.DMA,
        pltpu.SemaphoreType.DMA,
    ],
)(_emb_body)


def kernel(x, table):
    xt = jnp.swapaxes(x.astype(jnp.int32), 0, 1)
    xw = jnp.swapaxes(xt.reshape(S, NW, R_PER_W), 0, 1)
    out = _emb(xw.reshape(-1), table)
    return jnp.swapaxes(out, 0, 1)


# speedup vs baseline: 1.0299x; 1.0299x over previous
import functools

import jax
import jax.numpy as jnp
from jax import lax
from jax.experimental import pallas as pl
from jax.experimental.pallas import tpu as pltpu
from jax.experimental.pallas import tpu_sc as plsc

D = 128
R, S = 4096, 50
NC, NS = 2, 16
NW = NC * NS
RPW = R // NW
NBUF = 5
LA = 4


def _emb_body(xt_hbm, table_hbm, out_hbm, idx_v, b0, b1, b2, b3, b4,
              isem, g0, g1, g2, g3, g4, w0, w1, w2, w3, w4):
    wid = lax.axis_index("s") * NC + lax.axis_index("c")
    r0 = wid * RPW
    pltpu.async_copy(
        xt_hbm.at[pl.ds(wid * S * RPW, S * RPW)], idx_v, isem).wait()

    bufs = (b0, b1, b2, b3, b4)
    gsem = (g0, g1, g2, g3, g4)
    wsem = (w0, w1, w2, w3, w4)

    def issue_gather(i, k):
        pltpu.async_copy(
            table_hbm.at[idx_v.at[pl.ds(i * RPW, RPW)]], bufs[k], gsem[k])

    def wait_gather(k):
        pltpu.make_async_copy(
            table_hbm.at[pl.ds(0, RPW)], bufs[k], gsem[k]).wait()

    def issue_write(i, k):
        pltpu.async_copy(bufs[k], out_hbm.at[i, pl.ds(r0, RPW)], wsem[k])

    def wait_write(k):
        pltpu.make_async_copy(
            bufs[k], out_hbm.at[0, pl.ds(r0, RPW)], wsem[k]).wait()

    for i in range(LA):
        issue_gather(i, i)

    for k in range(NBUF):
        wait_gather(k)
        if k >= 1:
            wait_write(k - 1)
        if k + LA < 2 * NBUF - 1:
            issue_gather(k + LA, (k + LA) % NBUF)
        issue_write(k, k)

    @pl.loop(NBUF, S - NBUF, step=NBUF)
    def _steady(i0):
        for k in range(NBUF):
            wait_gather(k)
            wait_write((k + LA) % NBUF)
            issue_gather(i0 + k + LA, (k + LA) % NBUF)
            issue_write(i0 + k, k)

    for k in range(NBUF):
        i = S - NBUF + k
        wait_gather(k)
        if k == 0:
            wait_write((k + LA) % NBUF)
            issue_gather(i + LA, (i + LA) % NBUF)
        issue_write(i, k)
    for k in range(NBUF):
        wait_write(k)


_emb = functools.partial(
    pl.kernel,
    out_type=jax.ShapeDtypeStruct((S, R, D), jnp.float32),
    mesh=plsc.VectorSubcoreMesh(core_axis_name="c", subcore_axis_name="s"),
    scratch_types=[
        pltpu.VMEM((S * RPW,), jnp.int32),
        pltpu.VMEM((RPW, D), jnp.float32),
        pltpu.VMEM((RPW, D), jnp.float32),
        pltpu.VMEM((RPW, D), jnp.float32),
        pltpu.VMEM((RPW, D), jnp.float32),
        pltpu.VMEM((RPW, D), jnp.float32),
        pltpu.SemaphoreType.DMA,
        pltpu.SemaphoreType.DMA,
        pltpu.SemaphoreType.DMA,
        pltpu.SemaphoreType.DMA,
        pltpu.SemaphoreType.DMA,
        pltpu.SemaphoreType.DMA,
        pltpu.SemaphoreType.DMA,
        pltpu.SemaphoreType.DMA,
        pltpu.SemaphoreType.DMA,
        pltpu.SemaphoreType.DMA,
        pltpu.SemaphoreType.DMA,
    ],
)(_emb_body)


def kernel(x, table):
    xt = jnp.swapaxes(x.astype(jnp.int32), 0, 1)
    xw = jnp.swapaxes(xt.reshape(S, NW, RPW), 0, 1)
    out = _emb(xw.reshape(-1), table)
    return jnp.swapaxes(out, 0, 1)
